# baseline (speedup 1.0000x reference)
import functools

import jax
import jax.numpy as jnp
from jax.experimental import pallas as pl
from jax.experimental.pallas import tpu as pltpu

_R = 8
_NBUF = 8


def _margin_kernel(k_ref, pred_hbm, out_ref, bufs, tails, pk_acc, sems, semt,
                   *, B, C):
    C_al = (C // 128) * 128
    tw = C - C_al
    nchunks = B // _R
    lane = jax.lax.broadcasted_iota(jnp.int32, (1, 128), 1)
    tlane = jax.lax.broadcasted_iota(jnp.int32, (1, tw), 1)

    def start_copy(t, b):
        rows = pl.ds(t * _R, _R)
        pltpu.make_async_copy(
            pred_hbm.at[rows, pl.ds(0, C_al)], bufs.at[b], sems.at[b]).start()
        pltpu.make_async_copy(
            pred_hbm.at[rows, pl.ds(C_al, tw)], tails.at[b], semt.at[b]).start()

    for t in range(_NBUF):
        start_copy(t, t)

    def body(t, carry):
        b = jax.lax.rem(t, _NBUF)
        rows = pl.ds(t * _R, _R)
        pltpu.make_async_copy(
            pred_hbm.at[rows, pl.ds(0, C_al)], bufs.at[b], sems.at[b]).wait()
        pltpu.make_async_copy(
            pred_hbm.at[rows, pl.ds(C_al, tw)], tails.at[b], semt.at[b]).wait()

        for r in range(_R):
            c = k_ref[t * _R + r]

            def _bulk_rmw(c=c, r=r, b=b):
                c0 = (c // 128) * 128
                chunk = bufs[b, pl.ds(r, 1), pl.ds(c0, 128)]
                is_l = lane == (c - c0)
                pk_acc[pl.ds(r, 1), :] = jnp.where(is_l, chunk, -jnp.inf).max(
                    axis=1, keepdims=True)
                bufs[b, pl.ds(r, 1), pl.ds(c0, 128)] = jnp.where(
                    is_l, -jnp.inf, chunk)

            def _tail_rmw(c=c, r=r, b=b):
                chunk = tails[b, pl.ds(r, 1), :]
                is_l = tlane == (c - C_al)
                pk_acc[pl.ds(r, 1), :] = jnp.where(is_l, chunk, -jnp.inf).max(
                    axis=1, keepdims=True)
                tails[b, pl.ds(r, 1), :] = jnp.where(is_l, -jnp.inf, chunk)

            pl.when(c < C_al)(_bulk_rmw)
            pl.when(c >= C_al)(_tail_rmw)

        m = jnp.maximum(jnp.max(bufs[b], axis=1), jnp.max(tails[b], axis=1))
        out_ref[pl.ds(t * _R, _R), :] = m[:, None] - pk_acc[...]

        nxt = t + _NBUF

        @pl.when(nxt < nchunks)
        def _():
            start_copy(nxt, b)

        return carry

    jax.lax.fori_loop(0, nchunks, body, 0, unroll=False)


def kernel(prediction, k):
    B, C = prediction.shape
    k2 = k.astype(jnp.int32)
    C_al = (C // 128) * 128
    tw = C - C_al
    out = pl.pallas_call(
        functools.partial(_margin_kernel, B=B, C=C),
        in_specs=[
            pl.BlockSpec(memory_space=pltpu.SMEM),
            pl.BlockSpec(memory_space=pltpu.MemorySpace.HBM),
        ],
        out_specs=pl.BlockSpec(memory_space=pltpu.VMEM),
        out_shape=jax.ShapeDtypeStruct((B, 1), jnp.float32),
        scratch_shapes=[
            pltpu.VMEM((_NBUF, _R, C_al), jnp.float32),
            pltpu.VMEM((_NBUF, _R, tw), jnp.float32),
            pltpu.VMEM((_R, 1), jnp.float32),
            pltpu.SemaphoreType.DMA((_NBUF,)),
            pltpu.SemaphoreType.DMA((_NBUF,)),
        ],
    )(k2, prediction)
    return out.reshape(B)


# speedup vs baseline: 1.0077x; 1.0077x over previous
import functools

import jax
import jax.numpy as jnp
from jax import lax
from jax.experimental import pallas as pl
from jax.experimental.pallas import tpu as pltpu
from jax.experimental.pallas import tpu_sc as plsc

_NC = 2
_NS = 16
_NW = _NC * _NS
_WC = 3840
_NCH = 26
_REM0 = _NCH * _WC
_TAIL0 = _REM0 + 128


def _sc_margin(pred_hbm, tail_hbm, k_hbm, out_hbm,
               kv, b0, b1, rembuf, tailbuf, pkv, ov, accr, sems, semr, semt,
               *, B, C):
    rpw = B // _NW
    ngrp = rpw // 8
    tch = ngrp * _NCH
    wid = lax.axis_index("s") * _NC + lax.axis_index("c")
    row0 = wid * rpw
    pltpu.sync_copy(k_hbm.at[pl.ds(row0, rpw)], kv)

    bufs = (b0, b1)
    neg = jnp.full((16,), -jnp.inf, jnp.float32)
    lane = lax.iota(jnp.int32, 16)

    def kscal(rl):
        base = (rl // 16) * 16
        kvv = kv[pl.ds(base, 16)]
        sel = lane == jnp.full((16,), rl - base, jnp.int32)
        return jnp.max(jnp.where(sel, kvv, jnp.zeros((16,), jnp.int32)))

    def rmw(buf, r, cl, rl):
        j0 = (cl // 16) * 16
        v = buf[r, pl.ds(j0, 16)]
        il = lane == jnp.full((16,), cl - j0, jnp.int32)
        pkv[pl.ds(rl * 16, 16)] = jnp.where(il, v, neg)
        buf[r, pl.ds(j0, 16)] = jnp.where(il, neg, v)

    def start_main(t, b):
        gr0 = row0 + (t // _NCH) * 8
        col = (t % _NCH) * _WC
        pltpu.make_async_copy(
            pred_hbm.at[pl.ds(gr0, 8), pl.ds(col, _WC)], bufs[b], sems.at[b]
        ).start()

    start_main(0, 0)
    start_main(1, 1)

    def lg_max(buf, r):
        vs = [buf[r, pl.ds(h * 16, 16)] for h in range(8)]
        m01 = jnp.maximum(jnp.maximum(vs[0], vs[1]), jnp.maximum(vs[2], vs[3]))
        m23 = jnp.maximum(jnp.maximum(vs[4], vs[5]), jnp.maximum(vs[6], vs[7]))
        return jnp.maximum(m01, m23)

    @pl.loop(0, tch, step=2)
    def _chunks(g):
        for b in range(2):
            t = g + b
            buf = bufs[b]
            pltpu.make_async_copy(
                pred_hbm.at[pl.ds(row0, 8), pl.ds(0, _WC)], buf, sems.at[b]
            ).wait()
            rg = t // _NCH
            pos = t - rg * _NCH
            gr0 = row0 + rg * 8

            @pl.when(pos == 0)
            def _prime_group():
                pltpu.make_async_copy(
                    pred_hbm.at[pl.ds(gr0, 8), pl.ds(_REM0, 128)],
                    rembuf, semr).start()
                pltpu.make_async_copy(
                    tail_hbm.at[pl.ds(gr0, 8)], tailbuf, semt).start()
                for i in range(8):
                    accr[pl.ds(16 * i, 16)] = neg

            lo = pos * _WC
            for r in range(8):
                rl = rg * 8 + r
                kr = kscal(rl)

                @pl.when((kr >= lo) & (kr < lo + _WC))
                def _(r=r, kr=kr, rl=rl, lo=lo):
                    rmw(buf, r, kr - lo, rl)

            a = tuple(accr[pl.ds(16 * i, 16)] for i in range(8))

            def mb(tt, a):
                base = tt * 128
                out = []
                for r in range(8):
                    vs = [buf[r, pl.ds(base + h * 16, 16)] for h in range(8)]
                    m01 = jnp.maximum(jnp.maximum(vs[0], vs[1]),
                                      jnp.maximum(vs[2], vs[3]))
                    m23 = jnp.maximum(jnp.maximum(vs[4], vs[5]),
                                      jnp.maximum(vs[6], vs[7]))
                    out.append(jnp.maximum(a[r], jnp.maximum(m01, m23)))
                return tuple(out)

            a = lax.fori_loop(0, _WC // 128, mb, a)
            for i in range(8):
                accr[pl.ds(16 * i, 16)] = a[i]

            @pl.when(t + 2 < tch)
            def _prefetch():
                start_main(t + 2, b)

            @pl.when(pos == _NCH - 1)
            def _finalize():
                pltpu.make_async_copy(
                    pred_hbm.at[pl.ds(row0, 8), pl.ds(0, 128)],
                    rembuf, semr).wait()
                pltpu.make_async_copy(
                    pred_hbm.at[pl.ds(row0, 8), pl.ds(0, 128)],
                    tailbuf, semt).wait()
                for r in range(8):
                    rl = rg * 8 + r
                    kr = kscal(rl)

                    @pl.when((kr >= _REM0) & (kr < _TAIL0))
                    def _(r=r, kr=kr, rl=rl):
                        rmw(rembuf, r, kr - _REM0, rl)

                    @pl.when(kr >= _TAIL0)
                    def _(r=r, kr=kr, rl=rl):
                        rmw(tailbuf, r, kr - _TAIL0, rl)

                    m16 = jnp.maximum(accr[pl.ds(16 * r, 16)],
                                      jnp.maximum(lg_max(rembuf, r),
                                                  lg_max(tailbuf, r)))
                    margin = jnp.max(m16) - jnp.max(pkv[pl.ds(rl * 16, 16)])
                    rhi = (rl // 16) * 16
                    il = lane == jnp.full((16,), rl - rhi, jnp.int32)
                    ov[pl.ds(rhi, 16)] = jnp.where(
                        il, jnp.full((16,), margin), ov[pl.ds(rhi, 16)])

    pltpu.sync_copy(ov, out_hbm.at[pl.ds(row0, rpw)])


def kernel(prediction, k):
    B, C = prediction.shape
    k2 = k.astype(jnp.int32)
    tail = jnp.pad(prediction[:, _TAIL0:], ((0, 0), (0, 128 - (C - _TAIL0))),
                   constant_values=-jnp.inf)
    rpw = B // _NW
    mesh = plsc.VectorSubcoreMesh(core_axis_name="c", subcore_axis_name="s")
    out = pl.kernel(
        functools.partial(_sc_margin, B=B, C=C),
        out_type=jax.ShapeDtypeStruct((B,), jnp.float32),
        mesh=mesh,
        scratch_types=[
            pltpu.VMEM((rpw,), jnp.int32),
            pltpu.VMEM((8, _WC), jnp.float32),
            pltpu.VMEM((8, _WC), jnp.float32),
            pltpu.VMEM((8, 128), jnp.float32),
            pltpu.VMEM((8, 128), jnp.float32),
            pltpu.VMEM((rpw * 16,), jnp.float32),
            pltpu.VMEM((rpw,), jnp.float32),
            pltpu.VMEM((128,), jnp.float32),
            pltpu.SemaphoreType.DMA((2,)),
            pltpu.SemaphoreType.DMA,
            pltpu.SemaphoreType.DMA,
        ],
        compiler_params=pltpu.CompilerParams(use_tc_tiling_on_sc=True,
                                             needs_layout_passes=False),
    )(prediction, tail, k2)
    return out
